# baseline (speedup 1.0000x reference)
import functools

import jax
import jax.numpy as jnp
from jax import lax
from jax.experimental import pallas as pl
from jax.experimental.pallas import tpu as pltpu
from jax.experimental.pallas import tpu_sc as plsc

VOCAB = 1000000
MAXLEN = 200
DIM = 64
BATCH = 4096

NUM_CORES = 2
NUM_SUBCORES = 16
NUM_WORKERS = NUM_CORES * NUM_SUBCORES
SEQ_PER_WORKER = BATCH // NUM_WORKERS
LANES = 16
GATHER_SPLITS = ((0, 120), (120, 80))


def _add_pos(rows_v, pos_v, out_v):
    @pl.loop(0, MAXLEN)
    def _(i):
        for j in range(DIM // LANES):
            sl = pl.ds(j * LANES, LANES)
            out_v[i, sl] = rows_v[i, sl] + pos_v[i, sl]


def _body(x_hbm, tok_hbm, pos_hbm, out_hbm, pos_v, idx_v, rows_v, out_v,
          gsem, osem):
    wid = lax.axis_index("s") * NUM_CORES + lax.axis_index("c")
    seq_base = wid * SEQ_PER_WORKER

    pltpu.sync_copy(pos_hbm, pos_v)

    @pl.loop(0, SEQ_PER_WORKER)
    def _(g):
        seq = seq_base + g
        pltpu.sync_copy(x_hbm.at[seq], idx_v)
        for off, n in GATHER_SPLITS:
            sl = pl.ds(off, n)
            pltpu.async_copy(tok_hbm.at[idx_v.at[sl]], rows_v.at[sl], gsem)
        for off, n in GATHER_SPLITS:
            sl = pl.ds(off, n)
            pltpu.make_async_copy(tok_hbm.at[idx_v.at[sl]], rows_v.at[sl],
                                  gsem).wait()
        _add_pos(rows_v, pos_v, out_v)
        pltpu.async_copy(out_v, out_hbm.at[seq], osem).wait()


@jax.jit
def _sc_embed(x, token_table, pos_table):
    mesh = plsc.VectorSubcoreMesh(core_axis_name="c", subcore_axis_name="s")
    run = pl.kernel(
        _body,
        out_type=jax.ShapeDtypeStruct((BATCH, MAXLEN, DIM), jnp.float32),
        mesh=mesh,
        compiler_params=pltpu.CompilerParams(use_tc_tiling_on_sc=False),
        scratch_types=[
            pltpu.VMEM((MAXLEN, DIM), jnp.float32),
            pltpu.VMEM((MAXLEN,), jnp.int32),
            pltpu.VMEM((MAXLEN, DIM), jnp.float32),
            pltpu.VMEM((MAXLEN, DIM), jnp.float32),
            pltpu.SemaphoreType.DMA,
            pltpu.SemaphoreType.DMA,
        ],
    )
    return run(x, token_table, pos_table)


def kernel(x, token_table, pos_table):
    return _sc_embed(x.astype(jnp.int32), token_table, pos_table)


# speedup vs baseline: 1.1527x; 1.1527x over previous
import jax
import jax.numpy as jnp
from jax import lax
from jax.experimental import pallas as pl
from jax.experimental.pallas import tpu as pltpu
from jax.experimental.pallas import tpu_sc as plsc

VOCAB = 1000000
MAXLEN = 200
DIM = 64
BATCH = 4096

NUM_CORES = 2
NUM_SUBCORES = 16
NUM_WORKERS = NUM_CORES * NUM_SUBCORES
SEQ_PER_WORKER = BATCH // NUM_WORKERS
LANES = 16
GATHER_SPLITS = ((0, 120), (120, 80))


def _add_pos(rows_v, pos_v, out_v):
    @pl.loop(0, MAXLEN)
    def _(i):
        for j in range(DIM // LANES):
            sl = pl.ds(j * LANES, LANES)
            out_v[i, sl] = rows_v[i, sl] + pos_v[i, sl]


def _start_gather(tok_hbm, idx_v, rows_v, gsem):
    for off, n in GATHER_SPLITS:
        sl = pl.ds(off, n)
        pltpu.async_copy(tok_hbm.at[idx_v.at[sl]], rows_v.at[sl], gsem)


def _wait_gather(tok_hbm, idx_v, rows_v, gsem):
    for off, n in GATHER_SPLITS:
        sl = pl.ds(off, n)
        pltpu.make_async_copy(tok_hbm.at[idx_v.at[sl]], rows_v.at[sl],
                              gsem).wait()


def _body(x_hbm, tok_hbm, pos_hbm, out_hbm, pos_v, idx_v, rows_v, out_v,
          gsem, osem):
    wid = lax.axis_index("s") * NUM_CORES + lax.axis_index("c")
    seq_base = wid * SEQ_PER_WORKER

    pltpu.sync_copy(pos_hbm, pos_v)

    pltpu.sync_copy(x_hbm.at[seq_base], idx_v[0])
    _start_gather(tok_hbm, idx_v[0], rows_v[0], gsem[0])

    @pl.loop(0, SEQ_PER_WORKER, step=2)
    def _(g0):
        for p in range(2):
            q = 1 - p
            g = g0 + p
            seq = seq_base + g

            @pl.when(g + 1 < SEQ_PER_WORKER)
            def _():
                pltpu.sync_copy(x_hbm.at[seq + 1], idx_v[q])
                _start_gather(tok_hbm, idx_v[q], rows_v[q], gsem[q])

            _wait_gather(tok_hbm, idx_v[p], rows_v[p], gsem[p])

            @pl.when(g >= 2)
            def _():
                pltpu.make_async_copy(out_v[p], out_hbm.at[seq - 2],
                                      osem[p]).wait()

            _add_pos(rows_v[p], pos_v, out_v[p])
            pltpu.async_copy(out_v[p], out_hbm.at[seq], osem[p])

    for p in range(2):
        seq_last = seq_base + SEQ_PER_WORKER - 2 + p
        pltpu.make_async_copy(out_v[p], out_hbm.at[seq_last], osem[p]).wait()


@jax.jit
def _sc_embed(x, token_table, pos_table):
    mesh = plsc.VectorSubcoreMesh(core_axis_name="c", subcore_axis_name="s")
    run = pl.kernel(
        _body,
        out_type=jax.ShapeDtypeStruct((BATCH, MAXLEN, DIM), jnp.float32),
        mesh=mesh,
        compiler_params=pltpu.CompilerParams(use_tc_tiling_on_sc=False),
        scratch_types=[
            pltpu.VMEM((MAXLEN, DIM), jnp.float32),
            [pltpu.VMEM((MAXLEN,), jnp.int32)] * 2,
            [pltpu.VMEM((MAXLEN, DIM), jnp.float32)] * 2,
            [pltpu.VMEM((MAXLEN, DIM), jnp.float32)] * 2,
            [pltpu.SemaphoreType.DMA] * 2,
            [pltpu.SemaphoreType.DMA] * 2,
        ],
    )
    return run(x, token_table, pos_table)


def kernel(x, token_table, pos_table):
    return _sc_embed(x.astype(jnp.int32), token_table, pos_table)
